# Initial kernel scaffold; baseline (speedup 1.0000x reference)
import functools

import jax
import jax.numpy as jnp
from jax import lax
from jax.experimental import pallas as pl
from jax.experimental.pallas import tpu as pltpu
from jax.experimental.pallas import tpu_sc as plsc

_SCALING = 2.0

_B_TOT = 16384 * 50
_D = 64
_R = 16

_NC = 2
_NS = 16
_NW = _NC * _NS
_ROWS_PER_W = _B_TOT // _NW
_IDX_MINOR = 128
_CHUNK = 1024
_JS = _CHUNK // _IDX_MINOR
_N_CHUNKS = _ROWS_PER_W // _CHUNK
_IROWS_PER_W = _ROWS_PER_W // _IDX_MINOR


def _sc_gather(emb_hbm, a_hbm, idx_hbm, base_out, asel_out,
               idx_v, emb_v, a_v, sem_e, sem_a):
    wid = lax.axis_index("s") * _NC + lax.axis_index("c")

    def chunk_body(k, carry):
        row0 = wid * _ROWS_PER_W + k * _CHUNK
        irow0 = wid * _IROWS_PER_W + k * _JS
        pltpu.sync_copy(idx_hbm.at[pl.ds(irow0, _JS)], idx_v)
        handles = []
        for j in range(_JS):
            handles.append(pltpu.async_copy(
                emb_hbm.at[idx_v.at[j]],
                emb_v.at[pl.ds(j * _IDX_MINOR, _IDX_MINOR)], sem_e))
            handles.append(pltpu.async_copy(
                a_hbm.at[idx_v.at[j]],
                a_v.at[pl.ds(j * _IDX_MINOR, _IDX_MINOR)], sem_a))
        for h in handles:
            h.wait()
        pltpu.sync_copy(emb_v, base_out.at[pl.ds(row0, _CHUNK)])
        pltpu.sync_copy(a_v, asel_out.at[pl.ds(row0, _CHUNK)])
        return carry

    lax.fori_loop(0, _N_CHUNKS, chunk_body, 0)


_sc_gather_call = pl.kernel(
    _sc_gather,
    out_type=(
        jax.ShapeDtypeStruct((_B_TOT, _D), jnp.float32),
        jax.ShapeDtypeStruct((_B_TOT, _R), jnp.float32),
    ),
    mesh=plsc.VectorSubcoreMesh(core_axis_name="c", subcore_axis_name="s"),
    scratch_types=[
        pltpu.VMEM((_JS, _IDX_MINOR), jnp.int32),
        pltpu.VMEM((_CHUNK, _D), jnp.float32),
        pltpu.VMEM((_CHUNK, _R), jnp.float32),
        pltpu.SemaphoreType.DMA,
        pltpu.SemaphoreType.DMA,
    ],
)


_TC_BLK = 4096


def _tc_fuse(base_ref, a_ref, b_ref, out_ref):
    out_ref[...] = base_ref[...] + jnp.dot(
        a_ref[...], b_ref[...], preferred_element_type=jnp.float32) * _SCALING


_tc_fuse_call = pl.pallas_call(
    _tc_fuse,
    grid=(_B_TOT // _TC_BLK,),
    in_specs=[
        pl.BlockSpec((_TC_BLK, _D), lambda i: (i, 0)),
        pl.BlockSpec((_TC_BLK, _R), lambda i: (i, 0)),
        pl.BlockSpec((_R, _D), lambda i: (0, 0)),
    ],
    out_specs=pl.BlockSpec((_TC_BLK, _D), lambda i: (i, 0)),
    out_shape=jax.ShapeDtypeStruct((_B_TOT, _D), jnp.float32),
)


def kernel(inputs, embedding, lora_A, lora_B):
    batch, hist = inputs.shape
    idx2d = inputs.astype(jnp.int32).reshape(_B_TOT // _IDX_MINOR, _IDX_MINOR)
    base_sel, a_sel = _sc_gather_call(embedding, lora_A, idx2d)
    out = _tc_fuse_call(base_sel, a_sel, lora_B)
    return out.reshape(batch, hist, _D)


# speedup vs baseline: 6.8207x; 6.8207x over previous
import functools

import jax
import jax.numpy as jnp
from jax import lax
from jax.experimental import pallas as pl
from jax.experimental.pallas import tpu as pltpu
from jax.experimental.pallas import tpu_sc as plsc

_SCALING = 2.0

_B_TOT = 16384 * 50
_D = 64
_R = 16

_NC = 2
_NS = 16
_NW = _NC * _NS
_ROWS_PER_W = _B_TOT // _NW
_IDX_MINOR = 128
_CHUNK = 1024
_JS = _CHUNK // _IDX_MINOR
_N_CHUNKS = _ROWS_PER_W // _CHUNK
_IROWS_PER_W = _ROWS_PER_W // _IDX_MINOR


def _sc_gather(emb_hbm, a_hbm, idx_hbm, base_out, asel_out,
               idx_v, emb_v, a_v, sem_e, sem_a):
    wid = lax.axis_index("s") * _NC + lax.axis_index("c")

    def chunk_body(k, carry):
        row0 = wid * _ROWS_PER_W + k * _CHUNK
        irow0 = wid * _IROWS_PER_W + k * _JS
        pltpu.sync_copy(idx_hbm.at[pl.ds(irow0, _JS)], idx_v)
        handles = []
        for j in range(_JS):
            handles.append(pltpu.async_copy(
                emb_hbm.at[idx_v.at[j]],
                emb_v.at[pl.ds(j * _IDX_MINOR, _IDX_MINOR)], sem_e))
            handles.append(pltpu.async_copy(
                a_hbm.at[idx_v.at[j]],
                a_v.at[pl.ds(j * _IDX_MINOR, _IDX_MINOR)], sem_a))
        for h in handles:
            h.wait()
        pltpu.sync_copy(emb_v, base_out.at[pl.ds(row0, _CHUNK)])
        pltpu.sync_copy(a_v, asel_out.at[pl.ds(row0, _CHUNK)])
        return carry

    lax.fori_loop(0, _N_CHUNKS, chunk_body, 0)


_sc_gather_call = pl.kernel(
    _sc_gather,
    out_type=(
        jax.ShapeDtypeStruct((_B_TOT, _D), jnp.float32),
        jax.ShapeDtypeStruct((_B_TOT, _R), jnp.float32),
    ),
    mesh=plsc.VectorSubcoreMesh(core_axis_name="c", subcore_axis_name="s"),
    compiler_params=pltpu.CompilerParams(use_tc_tiling_on_sc=False),
    scratch_types=[
        pltpu.VMEM((_JS, _IDX_MINOR), jnp.int32),
        pltpu.VMEM((_CHUNK, _D), jnp.float32),
        pltpu.VMEM((_CHUNK, _R), jnp.float32),
        pltpu.SemaphoreType.DMA,
        pltpu.SemaphoreType.DMA,
    ],
)


_TC_BLK = 4096


def _tc_fuse(base_ref, a_ref, b_ref, out_ref):
    out_ref[...] = base_ref[...] + jnp.dot(
        a_ref[...], b_ref[...], preferred_element_type=jnp.float32) * _SCALING


_tc_fuse_call = pl.pallas_call(
    _tc_fuse,
    grid=(_B_TOT // _TC_BLK,),
    in_specs=[
        pl.BlockSpec((_TC_BLK, _D), lambda i: (i, 0)),
        pl.BlockSpec((_TC_BLK, _R), lambda i: (i, 0)),
        pl.BlockSpec((_R, _D), lambda i: (0, 0)),
    ],
    out_specs=pl.BlockSpec((_TC_BLK, _D), lambda i: (i, 0)),
    out_shape=jax.ShapeDtypeStruct((_B_TOT, _D), jnp.float32),
)


def kernel(inputs, embedding, lora_A, lora_B):
    batch, hist = inputs.shape
    idx2d = inputs.astype(jnp.int32).reshape(_B_TOT // _IDX_MINOR, _IDX_MINOR)
    base_sel, a_sel = _sc_gather_call(embedding, lora_A, idx2d)
    out = _tc_fuse_call(base_sel, a_sel, lora_B)
    return out.reshape(batch, hist, _D)


# speedup vs baseline: 7.3037x; 1.0708x over previous
import jax
import jax.numpy as jnp
from jax import lax
from jax.experimental import pallas as pl
from jax.experimental.pallas import tpu as pltpu
from jax.experimental.pallas import tpu_sc as plsc

_SCALING = 2.0

_B_TOT = 16384 * 50
_D = 64
_R = 16

_NC = 2
_NS = 16
_NW = _NC * _NS
_ROWS_PER_W = _B_TOT // _NW
_IDX_MINOR = 128
_CHUNK = 256
_JS = _CHUNK // _IDX_MINOR
_N_CHUNKS = _ROWS_PER_W // _CHUNK
_IROWS_PER_W = _ROWS_PER_W // _IDX_MINOR

_L = 16
_NJ = _D // _L

_BCAST_DN = lax.GatherDimensionNumbers(
    offset_dims=(), collapsed_slice_dims=(0,), start_index_map=(0,))


def _bcast(vec, r):
    idx = jnp.full((_L, 1), r, jnp.int32)
    return lax.gather(vec, idx, _BCAST_DN, slice_sizes=(1,),
                      mode=lax.GatherScatterMode.PROMISE_IN_BOUNDS)


def _sc_fused(emb_hbm, a_hbm, idx_hbm, b_hbm, out_hbm,
              idx_all, emb_v, a_v, out_v, b_v,
              sem_e0, sem_e1, sem_a0, sem_a1, sem_w0, sem_w1):
    sem_e = (sem_e0, sem_e1)
    sem_a = (sem_a0, sem_a1)
    sem_w = (sem_w0, sem_w1)
    wid = lax.axis_index("s") * _NC + lax.axis_index("c")

    pltpu.sync_copy(idx_hbm.at[pl.ds(wid * _IROWS_PER_W, _IROWS_PER_W)],
                    idx_all)
    pltpu.sync_copy(b_hbm, b_v)
    for r in range(_R):
        for j in range(_NJ):
            b_v[r, pl.ds(_L * j, _L)] = b_v[r, pl.ds(_L * j, _L)] * _SCALING

    def gather_descs(c, p):
        descs = []
        for j in range(_JS):
            irow = c * _JS + j
            dst = pl.ds(j * _IDX_MINOR, _IDX_MINOR)
            descs.append((emb_hbm.at[idx_all.at[irow]],
                          emb_v.at[p].at[dst], sem_e[p]))
            descs.append((a_hbm.at[idx_all.at[irow]],
                          a_v.at[p].at[dst], sem_a[p]))
        return descs

    def issue(c, p):
        for src, dst, sem in gather_descs(c, p):
            pltpu.async_copy(src, dst, sem)

    def drain(c, p):
        for src, dst, sem in gather_descs(c, p):
            pltpu.make_async_copy(src, dst, sem).wait()

    def compute(p):
        def half_pass(half):
            b_regs = [[b_v[8 * half + r, pl.ds(_L * j, _L)]
                       for j in range(_NJ)] for r in range(8)]
            src = emb_v if half == 0 else out_v

            def row(i, carry):
                a_row = a_v[p, i]
                accs = [src[p, i, pl.ds(_L * j, _L)] for j in range(_NJ)]
                for r in range(8):
                    ab = _bcast(a_row, 8 * half + r)
                    for j in range(_NJ):
                        accs[j] = accs[j] + ab * b_regs[r][j]
                for j in range(_NJ):
                    out_v[p, i, pl.ds(_L * j, _L)] = accs[j]
                return carry

            lax.fori_loop(0, _CHUNK, row, 0)

        half_pass(0)
        half_pass(1)

    issue(0, 0)
    issue(1, 1)

    def outer(k2, carry):
        for p in range(2):
            c = k2 * 2 + p
            drain(c, p)

            @pl.when(c >= 2)
            def _wait_prev_writeback():
                row0 = wid * _ROWS_PER_W + (c - 2) * _CHUNK
                pltpu.make_async_copy(
                    out_v.at[p], out_hbm.at[pl.ds(row0, _CHUNK)],
                    sem_w[p]).wait()

            compute(p)
            row0 = wid * _ROWS_PER_W + c * _CHUNK
            pltpu.async_copy(out_v.at[p], out_hbm.at[pl.ds(row0, _CHUNK)],
                             sem_w[p])

            @pl.when(c + 2 < _N_CHUNKS)
            def _prefetch_next():
                issue(c + 2, p)
        return carry

    lax.fori_loop(0, _N_CHUNKS // 2, outer, 0)

    for p in range(2):
        c_last = _N_CHUNKS - 2 + p
        row0 = wid * _ROWS_PER_W + c_last * _CHUNK
        pltpu.make_async_copy(out_v.at[p],
                              out_hbm.at[pl.ds(row0, _CHUNK)],
                              sem_w[p]).wait()


_sc_fused_call = pl.kernel(
    _sc_fused,
    out_type=jax.ShapeDtypeStruct((_B_TOT, _D), jnp.float32),
    mesh=plsc.VectorSubcoreMesh(core_axis_name="c", subcore_axis_name="s"),
    compiler_params=pltpu.CompilerParams(use_tc_tiling_on_sc=False),
    scratch_types=[
        pltpu.VMEM((_IROWS_PER_W, _IDX_MINOR), jnp.int32),
        pltpu.VMEM((2, _CHUNK, _D), jnp.float32),
        pltpu.VMEM((2, _CHUNK, _R), jnp.float32),
        pltpu.VMEM((2, _CHUNK, _D), jnp.float32),
        pltpu.VMEM((_R, _D), jnp.float32),
        pltpu.SemaphoreType.DMA,
        pltpu.SemaphoreType.DMA,
        pltpu.SemaphoreType.DMA,
        pltpu.SemaphoreType.DMA,
        pltpu.SemaphoreType.DMA,
        pltpu.SemaphoreType.DMA,
    ],
)


def kernel(inputs, embedding, lora_A, lora_B):
    batch, hist = inputs.shape
    idx2d = inputs.astype(jnp.int32).reshape(_B_TOT // _IDX_MINOR, _IDX_MINOR)
    out = _sc_fused_call(embedding, lora_A, idx2d, lora_B)
    return out.reshape(batch, hist, _D)


# speedup vs baseline: 7.4047x; 1.0138x over previous
import jax
import jax.numpy as jnp
from jax import lax
from jax.experimental import pallas as pl
from jax.experimental.pallas import tpu as pltpu
from jax.experimental.pallas import tpu_sc as plsc

_SCALING = 2.0

_BATCH = 16384
_HIST = 50
_D = 64
_R = 16

_NC = 2
_NS = 16
_NW = _NC * _NS
_IROWS_PER_W = _BATCH // _NW
_CI = 4
_CHUNK = _CI * _HIST
_N_CHUNKS = _IROWS_PER_W // _CI

_L = 16
_NJ = _D // _L

_BCAST_DN = lax.GatherDimensionNumbers(
    offset_dims=(), collapsed_slice_dims=(0,), start_index_map=(0,))


def _bcast(vec, r):
    idx = jnp.full((_L, 1), r, jnp.int32)
    return lax.gather(vec, idx, _BCAST_DN, slice_sizes=(1,),
                      mode=lax.GatherScatterMode.PROMISE_IN_BOUNDS)


def _sc_fused(emb_hbm, a_hbm, idx_hbm, b_hbm, out_hbm,
              idx_all, emb_v, a_v, out_v, b_v,
              sem_e0, sem_e1, sem_a0, sem_a1, sem_w0, sem_w1):
    sem_e = (sem_e0, sem_e1)
    sem_a = (sem_a0, sem_a1)
    sem_w = (sem_w0, sem_w1)
    wid = lax.axis_index("s") * _NC + lax.axis_index("c")
    irow_base = wid * _IROWS_PER_W

    pltpu.sync_copy(idx_hbm.at[pl.ds(irow_base, _IROWS_PER_W)], idx_all)
    pltpu.sync_copy(b_hbm, b_v)
    for r in range(_R):
        for j in range(_NJ):
            b_v[r, pl.ds(_L * j, _L)] = b_v[r, pl.ds(_L * j, _L)] * _SCALING

    def gather_descs(c, p):
        descs = []
        for j in range(_CI):
            dst = pl.ds(j * _HIST, _HIST)
            descs.append((emb_hbm.at[idx_all.at[c * _CI + j]],
                          emb_v.at[p].at[dst], sem_e[p]))
            descs.append((a_hbm.at[idx_all.at[c * _CI + j]],
                          a_v.at[p].at[dst], sem_a[p]))
        return descs

    def issue(c, p):
        for src, dst, sem in gather_descs(c, p):
            pltpu.async_copy(src, dst, sem)

    def drain(c, p):
        for src, dst, sem in gather_descs(c, p):
            pltpu.make_async_copy(src, dst, sem).wait()

    def compute(p):
        b_regs = [[b_v[r, pl.ds(_L * j, _L)] for j in range(_NJ)]
                  for r in range(_R)]
        for q in range(_CI):
            @plsc.parallel_loop(0, _HIST, unroll=2)
            def _row(rr):
                i = q * _HIST + rr
                a_row = a_v[p, i]
                accs = [emb_v[p, i, pl.ds(_L * j, _L)] for j in range(_NJ)]
                for r in range(_R):
                    ab = _bcast(a_row, r)
                    for j in range(_NJ):
                        accs[j] = accs[j] + ab * b_regs[r][j]
                for j in range(_NJ):
                    out_v[p, q, rr, pl.ds(_L * j, _L)] = accs[j]

    def wb_desc(c, p):
        row0 = irow_base + c * _CI
        return (out_v.at[p], out_hbm.at[pl.ds(row0, _CI)], sem_w[p])

    issue(0, 0)
    issue(1, 1)

    def outer(k2, carry):
        for p in range(2):
            c = k2 * 2 + p
            drain(c, p)

            @pl.when(c >= 2)
            def _wait_prev_writeback():
                src, dst, sem = wb_desc(c - 2, p)
                pltpu.make_async_copy(src, dst, sem).wait()

            compute(p)
            src, dst, sem = wb_desc(c, p)
            pltpu.async_copy(src, dst, sem)

            @pl.when(c + 2 < _N_CHUNKS)
            def _prefetch_next():
                issue(c + 2, p)
        return carry

    lax.fori_loop(0, _N_CHUNKS // 2, outer, 0)

    for p in range(2):
        src, dst, sem = wb_desc(_N_CHUNKS - 2 + p, p)
        pltpu.make_async_copy(src, dst, sem).wait()


_sc_fused_call = pl.kernel(
    _sc_fused,
    out_type=jax.ShapeDtypeStruct((_BATCH, _HIST, _D), jnp.float32),
    mesh=plsc.VectorSubcoreMesh(core_axis_name="c", subcore_axis_name="s"),
    compiler_params=pltpu.CompilerParams(use_tc_tiling_on_sc=False),
    scratch_types=[
        pltpu.VMEM((_IROWS_PER_W, _HIST), jnp.int32),
        pltpu.VMEM((2, _CHUNK, _D), jnp.float32),
        pltpu.VMEM((2, _CHUNK, _R), jnp.float32),
        pltpu.VMEM((2, _CI, _HIST, _D), jnp.float32),
        pltpu.VMEM((_R, _D), jnp.float32),
        pltpu.SemaphoreType.DMA,
        pltpu.SemaphoreType.DMA,
        pltpu.SemaphoreType.DMA,
        pltpu.SemaphoreType.DMA,
        pltpu.SemaphoreType.DMA,
        pltpu.SemaphoreType.DMA,
    ],
)


def kernel(inputs, embedding, lora_A, lora_B):
    return _sc_fused_call(embedding, lora_A, inputs.astype(jnp.int32), lora_B)


# speedup vs baseline: 7.8177x; 1.0558x over previous
import jax
import jax.numpy as jnp
from jax import lax
from jax.experimental import pallas as pl
from jax.experimental.pallas import tpu as pltpu
from jax.experimental.pallas import tpu_sc as plsc

_SCALING = 2.0

_V = 1000000
_BATCH = 16384
_HIST = 50
_D = 64
_R = 16
_W = 128

_NC = 2
_NS = 16
_NW = _NC * _NS
_IROWS_PER_W = _BATCH // _NW
_CI = 4
_CHUNK = _CI * _HIST
_N_CHUNKS = _IROWS_PER_W // _CI

_L = 16
_NJ = _D // _L

_BCAST_DN = lax.GatherDimensionNumbers(
    offset_dims=(), collapsed_slice_dims=(0,), start_index_map=(0,))


def _bcast(vec, r):
    idx = jnp.full((_L, 1), r, jnp.int32)
    return lax.gather(vec, idx, _BCAST_DN, slice_sizes=(1,),
                      mode=lax.GatherScatterMode.PROMISE_IN_BOUNDS)


_PACK_BLK = 8000


def _pack_body(emb_ref, a_ref, out_ref):
    z = jnp.zeros((_PACK_BLK, _W - _D - _R), jnp.float32)
    out_ref[...] = jnp.concatenate(
        [emb_ref[...], a_ref[...] * _SCALING, z], axis=-1)


_pack_call = pl.pallas_call(
    _pack_body,
    grid=(_V // _PACK_BLK,),
    in_specs=[
        pl.BlockSpec((_PACK_BLK, _D), lambda i: (i, 0)),
        pl.BlockSpec((_PACK_BLK, _R), lambda i: (i, 0)),
    ],
    out_specs=pl.BlockSpec((_PACK_BLK, _W), lambda i: (i, 0)),
    out_shape=jax.ShapeDtypeStruct((_V, _W), jnp.float32),
)

_IDX_BLK = 2048


def _idx_pad_body(idx_ref, out_ref):
    z = jnp.zeros((_IDX_BLK, _W - _HIST), jnp.int32)
    out_ref[...] = jnp.concatenate([idx_ref[...], z], axis=-1)


_idx_pad_call = pl.pallas_call(
    _idx_pad_body,
    grid=(_BATCH // _IDX_BLK,),
    in_specs=[pl.BlockSpec((_IDX_BLK, _HIST), lambda i: (i, 0))],
    out_specs=pl.BlockSpec((_IDX_BLK, _W), lambda i: (i, 0)),
    out_shape=jax.ShapeDtypeStruct((_BATCH, _W), jnp.int32),
)


def _b_pad_body(b_ref, out_ref):
    z = jnp.zeros((_R, _W - _D), jnp.float32)
    out_ref[...] = jnp.concatenate([b_ref[...], z], axis=-1)


_b_pad_call = pl.pallas_call(
    _b_pad_body,
    in_specs=[pl.BlockSpec((_R, _D), lambda: (0, 0))],
    out_specs=pl.BlockSpec((_R, _W), lambda: (0, 0)),
    out_shape=jax.ShapeDtypeStruct((_R, _W), jnp.float32),
)


def _sc_fused(tab_hbm, idx_hbm, b_hbm, out_hbm,
              idx_v, g_v, out_v, b_v,
              sem_g0, sem_g1, sem_w0, sem_w1,
              sem_i0, sem_i1, sem_i2, sem_i3):
    sem_g = (sem_g0, sem_g1)
    sem_w = (sem_w0, sem_w1)
    sem_i = (sem_i0, sem_i1, sem_i2, sem_i3)
    wid = lax.axis_index("s") * _NC + lax.axis_index("c")
    irow_base = wid * _IROWS_PER_W

    pltpu.sync_copy(b_hbm, b_v)

    def idx_desc(c, s):
        return (idx_hbm.at[pl.ds(irow_base + c * _CI, _CI)],
                idx_v.at[s], sem_i[s])

    def gather_descs(c, s, p):
        descs = []
        for j in range(_CI):
            descs.append((tab_hbm.at[idx_v.at[s, j, pl.ds(0, _HIST)]],
                          g_v.at[p].at[pl.ds(j * _HIST, _HIST)], sem_g[p]))
        return descs

    def issue(c, s, p):
        for src, dst, sem in gather_descs(c, s, p):
            pltpu.async_copy(src, dst, sem)

    def drain(c, s, p):
        for src, dst, sem in gather_descs(c, s, p):
            pltpu.make_async_copy(src, dst, sem).wait()

    def compute(p):
        b_regs = [[b_v[r, pl.ds(_L * j, _L)] for j in range(_NJ)]
                  for r in range(_R)]
        for q in range(_CI):
            @plsc.parallel_loop(0, _HIST, unroll=2)
            def _row(rr):
                i = q * _HIST + rr
                a_row = g_v[p, i, pl.ds(_D, _R)]
                accs = [g_v[p, i, pl.ds(_L * j, _L)] for j in range(_NJ)]
                for r in range(_R):
                    ab = _bcast(a_row, r)
                    for j in range(_NJ):
                        accs[j] = accs[j] + ab * b_regs[r][j]
                for j in range(_NJ):
                    out_v[p, q, rr, pl.ds(_L * j, _L)] = accs[j]

    def wb_desc(c, p):
        row0 = irow_base + c * _CI
        return (out_v.at[p], out_hbm.at[pl.ds(row0, _CI)], sem_w[p])

    for c in range(4):
        src, dst, sem = idx_desc(c, c)
        pltpu.async_copy(src, dst, sem)
    for c in range(2):
        src, dst, sem = idx_desc(c, c)
        pltpu.make_async_copy(src, dst, sem).wait()
        issue(c, c, c)

    def outer(k4, carry):
        for u in range(4):
            c = k4 * 4 + u
            p = u % 2
            drain(c, u, p)

            @pl.when(c >= 2)
            def _wait_prev_writeback():
                src, dst, sem = wb_desc(c - 2, p)
                pltpu.make_async_copy(src, dst, sem).wait()

            compute(p)
            src, dst, sem = wb_desc(c, p)
            pltpu.async_copy(src, dst, sem)

            @pl.when(c + 2 < _N_CHUNKS)
            def _prefetch_next():
                si, di, smi = idx_desc(c + 2, (u + 2) % 4)
                pltpu.make_async_copy(si, di, smi).wait()
                issue(c + 2, (u + 2) % 4, p)

            @pl.when(c + 4 < _N_CHUNKS)
            def _stage_next_idx():
                si, di, smi = idx_desc(c + 4, u)
                pltpu.async_copy(si, di, smi)
        return carry

    lax.fori_loop(0, _N_CHUNKS // 4, outer, 0)

    for p in range(2):
        src, dst, sem = wb_desc(_N_CHUNKS - 2 + p, p)
        pltpu.make_async_copy(src, dst, sem).wait()


_sc_fused_call = pl.kernel(
    _sc_fused,
    out_type=jax.ShapeDtypeStruct((_BATCH, _HIST, _D), jnp.float32),
    mesh=plsc.VectorSubcoreMesh(core_axis_name="c", subcore_axis_name="s"),
    compiler_params=pltpu.CompilerParams(use_tc_tiling_on_sc=True),
    scratch_types=[
        pltpu.VMEM((4, _CI, _W), jnp.int32),
        pltpu.VMEM((2, _CHUNK, _W), jnp.float32),
        pltpu.VMEM((2, _CI, _HIST, _D), jnp.float32),
        pltpu.VMEM((_R, _W), jnp.float32),
        pltpu.SemaphoreType.DMA,
        pltpu.SemaphoreType.DMA,
        pltpu.SemaphoreType.DMA,
        pltpu.SemaphoreType.DMA,
        pltpu.SemaphoreType.DMA,
        pltpu.SemaphoreType.DMA,
        pltpu.SemaphoreType.DMA,
        pltpu.SemaphoreType.DMA,
    ],
)


def kernel(inputs, embedding, lora_A, lora_B):
    packed = _pack_call(embedding, lora_A)
    idx_pad = _idx_pad_call(inputs.astype(jnp.int32))
    b_pad = _b_pad_call(lora_B)
    return _sc_fused_call(packed, idx_pad, b_pad)


# speedup vs baseline: 8.7797x; 1.1231x over previous
import jax
import jax.numpy as jnp
from jax import lax
from jax.experimental import pallas as pl
from jax.experimental.pallas import tpu as pltpu
from jax.experimental.pallas import tpu_sc as plsc

_SCALING = 2.0

_V = 1000000
_BATCH = 16384
_HIST = 50
_D = 64
_R = 16
_W = 128

_NC = 2
_NS = 16
_NW = _NC * _NS
_IROWS_PER_W = _BATCH // _NW
_CI = 4
_CHUNK = _CI * _HIST
_N_CHUNKS = _IROWS_PER_W // _CI

_L = 16
_NJ = _D // _L

_BCAST_DN = lax.GatherDimensionNumbers(
    offset_dims=(), collapsed_slice_dims=(0,), start_index_map=(0,))


def _bcast(vec, r):
    idx = jnp.full((_L, 1), r, jnp.int32)
    return lax.gather(vec, idx, _BCAST_DN, slice_sizes=(1,),
                      mode=lax.GatherScatterMode.PROMISE_IN_BOUNDS)


def _sc_fused(tab_hbm, idx_hbm, b_hbm, out_hbm,
              idx_v, g_v, out_v, b_v,
              sem_g0, sem_g1, sem_w0, sem_w1,
              sem_i0, sem_i1, sem_i2, sem_i3):
    sem_g = (sem_g0, sem_g1)
    sem_w = (sem_w0, sem_w1)
    sem_i = (sem_i0, sem_i1, sem_i2, sem_i3)
    wid = lax.axis_index("s") * _NC + lax.axis_index("c")
    irow_base = wid * _IROWS_PER_W

    pltpu.sync_copy(b_hbm, b_v)
    for r in range(_R):
        for j in range(_NJ):
            b_v[r, pl.ds(_L * j, _L)] = b_v[r, pl.ds(_L * j, _L)] * _SCALING

    def idx_desc(c, s):
        return (idx_hbm.at[pl.ds(irow_base + c * _CI, _CI)],
                idx_v.at[s], sem_i[s])

    def gather_descs(c, s, p):
        descs = []
        for j in range(_CI):
            descs.append((tab_hbm.at[idx_v.at[s, j, pl.ds(0, _HIST)]],
                          g_v.at[p].at[pl.ds(j * _HIST, _HIST)], sem_g[p]))
        return descs

    def issue(c, s, p):
        for src, dst, sem in gather_descs(c, s, p):
            pltpu.async_copy(src, dst, sem)

    def drain(c, s, p):
        for src, dst, sem in gather_descs(c, s, p):
            pltpu.make_async_copy(src, dst, sem).wait()

    def compute(p):
        b_regs = [[b_v[r, pl.ds(_L * j, _L)] for j in range(_NJ)]
                  for r in range(_R)]
        for q in range(_CI):
            @plsc.parallel_loop(0, _HIST, unroll=2)
            def _row(rr):
                i = q * _HIST + rr
                a_row = g_v[p, i, pl.ds(_D, _R)]
                accs = [g_v[p, i, pl.ds(_L * j, _L)] for j in range(_NJ)]
                for r in range(_R):
                    ab = _bcast(a_row, r)
                    for j in range(_NJ):
                        accs[j] = accs[j] + ab * b_regs[r][j]
                for j in range(_NJ):
                    out_v[p, q, rr, pl.ds(_L * j, _L)] = accs[j]

    def wb_desc(c, p):
        row0 = irow_base + c * _CI
        return (out_v.at[p], out_hbm.at[pl.ds(row0, _CI)], sem_w[p])

    for c in range(4):
        src, dst, sem = idx_desc(c, c)
        pltpu.async_copy(src, dst, sem)
    for c in range(2):
        src, dst, sem = idx_desc(c, c)
        pltpu.make_async_copy(src, dst, sem).wait()
        issue(c, c, c)

    def outer(k4, carry):
        for u in range(4):
            c = k4 * 4 + u
            p = u % 2
            drain(c, u, p)

            @pl.when(c >= 2)
            def _wait_prev_writeback():
                src, dst, sem = wb_desc(c - 2, p)
                pltpu.make_async_copy(src, dst, sem).wait()

            compute(p)
            src, dst, sem = wb_desc(c, p)
            pltpu.async_copy(src, dst, sem)

            @pl.when(c + 2 < _N_CHUNKS)
            def _prefetch_next():
                si, di, smi = idx_desc(c + 2, (u + 2) % 4)
                pltpu.make_async_copy(si, di, smi).wait()
                issue(c + 2, (u + 2) % 4, p)

            @pl.when(c + 4 < _N_CHUNKS)
            def _stage_next_idx():
                si, di, smi = idx_desc(c + 4, u)
                pltpu.async_copy(si, di, smi)
        return carry

    lax.fori_loop(0, _N_CHUNKS // 4, outer, 0)

    for p in range(2):
        src, dst, sem = wb_desc(_N_CHUNKS - 2 + p, p)
        pltpu.make_async_copy(src, dst, sem).wait()


_sc_fused_call = pl.kernel(
    _sc_fused,
    out_type=jax.ShapeDtypeStruct((_BATCH, _HIST, _D), jnp.float32),
    mesh=plsc.VectorSubcoreMesh(core_axis_name="c", subcore_axis_name="s"),
    compiler_params=pltpu.CompilerParams(use_tc_tiling_on_sc=True),
    scratch_types=[
        pltpu.VMEM((4, _CI, _W), jnp.int32),
        pltpu.VMEM((2, _CHUNK, _W), jnp.float32),
        pltpu.VMEM((2, _CI, _HIST, _D), jnp.float32),
        pltpu.VMEM((_R, _W), jnp.float32),
        pltpu.SemaphoreType.DMA,
        pltpu.SemaphoreType.DMA,
        pltpu.SemaphoreType.DMA,
        pltpu.SemaphoreType.DMA,
        pltpu.SemaphoreType.DMA,
        pltpu.SemaphoreType.DMA,
        pltpu.SemaphoreType.DMA,
        pltpu.SemaphoreType.DMA,
    ],
)


def kernel(inputs, embedding, lora_A, lora_B):
    packed = jnp.concatenate(
        [embedding, lora_A,
         jnp.zeros((_V, _W - _D - _R), jnp.float32)], axis=-1)
    idx_pad = jnp.pad(inputs.astype(jnp.int32),
                      ((0, 0), (0, _W - _HIST)))
    b_pad = jnp.pad(lora_B, ((0, 0), (0, _W - _D)))
    return _sc_fused_call(packed, idx_pad, b_pad)
